# baseline (speedup 1.0000x reference)
import functools

import jax
import jax.numpy as jnp
from jax import lax
from jax.experimental import pallas as pl
from jax.experimental.pallas import tpu as pltpu
from jax.experimental.pallas import tpu_sc as plsc

VOCAB = 100000
VOCAB_PAD = 100096
EMBED_DIM = 64
BATCH = 4096
MAX_LEN = 200

NUM_WORKERS = 32
D_PER_WORKER = EMBED_DIM // NUM_WORKERS

_MESH = plsc.VectorSubcoreMesh(core_axis_name="c", subcore_axis_name="s")


@functools.partial(
    pl.kernel,
    mesh=_MESH,
    out_type=jax.ShapeDtypeStruct((MAX_LEN, 8, 32, 8, 128), jnp.float32),
    scratch_types=[
        pltpu.VMEM((782, 128), jnp.float32),
        pltpu.VMEM((32, 128), jnp.int32),
        pltpu.VMEM((32, 128), jnp.int32),
        pltpu.VMEM((32, 128), jnp.float32),
        pltpu.VMEM((32, 128), jnp.float32),
        pltpu.SemaphoreType.DMA,
        pltpu.SemaphoreType.DMA,
        pltpu.SemaphoreType.DMA,
        pltpu.SemaphoreType.DMA,
    ],
    compiler_params=pltpu.CompilerParams(
        use_tc_tiling_on_sc=False, needs_layout_passes=False
    ),
)
def _lookup_t(
    idx_hbm, table_hbm, out_hbm,
    row_v, ib0, ib1, ob0, ob1,
    si0, si1, so0, so1,
):
    cid = lax.axis_index("c")
    sid = lax.axis_index("s")
    wid = sid * 2 + cid

    ibufs = (ib0, ib1)
    isems = (si0, si1)
    obufs = (ob0, ob1)
    osems = (so0, so1)

    def idx_src(l):
        return idx_hbm.at[l // 8, :, l % 8, :]

    def out_dst(l, dp, dq):
        return out_hbm.at[l, dp, :, dq]

    for k in range(D_PER_WORKER):
        d = wid * D_PER_WORKER + k
        dp, dq = d // 8, d % 8
        pltpu.sync_copy(table_hbm.at[dp, :, dq], row_v)

        pltpu.async_copy(idx_src(0), ib0, si0)
        pltpu.async_copy(idx_src(1), ib1, si1)

        def m_body(m, carry, dp=dp, dq=dq):
            for b in range(2):
                l = 2 * m + b
                ib, isem = ibufs[b], isems[b]
                ob, osem = obufs[b], osems[b]
                pltpu.make_async_copy(idx_src(l), ib, isem).wait()
                @pl.when(m > 0)
                def _():
                    pltpu.make_async_copy(
                        ob, out_dst(l - 2, dp, dq), osem
                    ).wait()

                def g_body(g, c):
                    for j in range(8):
                        v16 = ib[g, pl.ds(j * 16, 16)]
                        h16 = lax.shift_right_logical(v16, 7)
                        l16 = lax.bitwise_and(v16, 127)
                        ob[g, pl.ds(j * 16, 16)] = plsc.load_gather(
                            row_v, [h16, l16]
                        )
                    return c

                lax.fori_loop(0, 32, g_body, 0, unroll=2)
                pltpu.async_copy(ob, out_dst(l, dp, dq), osem)

                @pl.when(l + 2 < MAX_LEN)
                def _():
                    pltpu.async_copy(idx_src(l + 2), ib, isem)

            return carry

        lax.fori_loop(0, MAX_LEN // 2, m_body, 0)
        pltpu.make_async_copy(ob0, out_dst(MAX_LEN - 2, dp, dq), so0).wait()
        pltpu.make_async_copy(ob1, out_dst(MAX_LEN - 1, dp, dq), so1).wait()


def kernel(indices, table):
    idx4 = indices.T.reshape(25, 8, 32, 128).transpose(0, 2, 1, 3)
    table_t = jnp.pad(table.T, ((0, 0), (0, VOCAB_PAD - VOCAB)))
    table4 = table_t.reshape(8, 8, 782, 128).transpose(0, 2, 1, 3)
    out5 = _lookup_t(idx4, table4)
    out = out5.transpose(2, 4, 0, 1, 3).reshape(BATCH, MAX_LEN, EMBED_DIM)
    return out


# speedup vs baseline: 2.6538x; 2.6538x over previous
import functools

import jax
import jax.numpy as jnp
from jax import lax
from jax.experimental import pallas as pl
from jax.experimental.pallas import tpu as pltpu
from jax.experimental.pallas import tpu_sc as plsc

VOCAB = 100000
VOCAB_PAD = 100096
EMBED_DIM = 64
BATCH = 4096
MAX_LEN = 200

NUM_WORKERS = 32
D_PER_WORKER = EMBED_DIM // NUM_WORKERS

_MESH = plsc.VectorSubcoreMesh(core_axis_name="c", subcore_axis_name="s")


@functools.partial(
    pl.kernel,
    mesh=_MESH,
    out_type=jax.ShapeDtypeStruct((MAX_LEN, 8, 32, 8, 128), jnp.float32),
    scratch_types=[
        pltpu.VMEM((782, 128), jnp.float32),
        pltpu.VMEM((32, 128), jnp.int32),
        pltpu.VMEM((32, 128), jnp.int32),
        pltpu.VMEM((32, 128), jnp.float32),
        pltpu.VMEM((32, 128), jnp.float32),
        pltpu.SemaphoreType.DMA,
        pltpu.SemaphoreType.DMA,
        pltpu.SemaphoreType.DMA,
        pltpu.SemaphoreType.DMA,
    ],
    compiler_params=pltpu.CompilerParams(
        use_tc_tiling_on_sc=False, needs_layout_passes=False
    ),
)
def _lookup_t(
    idx_hbm, table_hbm, out_hbm,
    row_v, ib0, ib1, ob0, ob1,
    si0, si1, so0, so1,
):
    cid = lax.axis_index("c")
    sid = lax.axis_index("s")
    wid = sid * 2 + cid

    ibufs = (ib0, ib1)
    isems = (si0, si1)
    obufs = (ob0, ob1)
    osems = (so0, so1)

    def idx_src(l):
        return idx_hbm.at[l // 8, :, l % 8, :]

    def out_dst(l, dp, dq):
        return out_hbm.at[l, dp, :, dq]

    for k in range(D_PER_WORKER):
        d = wid * D_PER_WORKER + k
        dp, dq = d // 8, d % 8
        pltpu.sync_copy(table_hbm.at[dp, :, dq], row_v)

        pltpu.async_copy(idx_src(0), ib0, si0)
        pltpu.async_copy(idx_src(1), ib1, si1)

        def m_body(m, carry, dp=dp, dq=dq):
            for b in range(2):
                l = 2 * m + b
                ib, isem = ibufs[b], isems[b]
                ob, osem = obufs[b], osems[b]
                pltpu.make_async_copy(idx_src(l), ib, isem).wait()
                @pl.when(m > 0)
                def _():
                    pltpu.make_async_copy(
                        ob, out_dst(l - 2, dp, dq), osem
                    ).wait()

                @plsc.parallel_loop(0, 32, unroll=2)
                def _(g, ib=ib, ob=ob):
                    for j in range(8):
                        v16 = ib[g, pl.ds(j * 16, 16)]
                        h16 = lax.shift_right_logical(v16, 7)
                        l16 = lax.bitwise_and(v16, 127)
                        ob[g, pl.ds(j * 16, 16)] = plsc.load_gather(
                            row_v, [h16, l16]
                        )
                pltpu.async_copy(ob, out_dst(l, dp, dq), osem)

                @pl.when(l + 2 < MAX_LEN)
                def _():
                    pltpu.async_copy(idx_src(l + 2), ib, isem)

            return carry

        lax.fori_loop(0, MAX_LEN // 2, m_body, 0)
        pltpu.make_async_copy(ob0, out_dst(MAX_LEN - 2, dp, dq), so0).wait()
        pltpu.make_async_copy(ob1, out_dst(MAX_LEN - 1, dp, dq), so1).wait()


def kernel(indices, table):
    idx4 = indices.T.reshape(25, 8, 32, 128).transpose(0, 2, 1, 3)
    table_t = jnp.pad(table.T, ((0, 0), (0, VOCAB_PAD - VOCAB)))
    table4 = table_t.reshape(8, 8, 782, 128).transpose(0, 2, 1, 3)
    out5 = _lookup_t(idx4, table4)
    out = out5.transpose(2, 4, 0, 1, 3).reshape(BATCH, MAX_LEN, EMBED_DIM)
    return out


# speedup vs baseline: 2.6716x; 1.0067x over previous
import functools

import jax
import jax.numpy as jnp
from jax import lax
from jax.experimental import pallas as pl
from jax.experimental.pallas import tpu as pltpu
from jax.experimental.pallas import tpu_sc as plsc

VOCAB = 100000
VOCAB_PAD = 100096
EMBED_DIM = 64
BATCH = 4096
MAX_LEN = 200

NUM_WORKERS = 32
D_PER_WORKER = EMBED_DIM // NUM_WORKERS

_MESH = plsc.VectorSubcoreMesh(core_axis_name="c", subcore_axis_name="s")


@functools.partial(
    pl.kernel,
    mesh=_MESH,
    out_type=jax.ShapeDtypeStruct((MAX_LEN, 8, 32, 8, 128), jnp.float32),
    scratch_types=[
        pltpu.VMEM((782, 128), jnp.float32),
        pltpu.VMEM((32, 128), jnp.int32),
        pltpu.VMEM((32, 128), jnp.int32),
        pltpu.VMEM((32, 128), jnp.float32),
        pltpu.VMEM((32, 128), jnp.float32),
        pltpu.SemaphoreType.DMA,
        pltpu.SemaphoreType.DMA,
        pltpu.SemaphoreType.DMA,
        pltpu.SemaphoreType.DMA,
    ],
    compiler_params=pltpu.CompilerParams(
        use_tc_tiling_on_sc=False, needs_layout_passes=False
    ),
)
def _lookup_t(
    idx_hbm, table_hbm, out_hbm,
    row_v, ib0, ib1, ob0, ob1,
    si0, si1, so0, so1,
):
    cid = lax.axis_index("c")
    sid = lax.axis_index("s")
    wid = sid * 2 + cid

    ibufs = (ib0, ib1)
    isems = (si0, si1)
    obufs = (ob0, ob1)
    osems = (so0, so1)

    def idx_src(l):
        return idx_hbm.at[l // 8, :, l % 8, :]

    def out_dst(l, dp, dq):
        return out_hbm.at[l, dp, :, dq]

    for k in range(D_PER_WORKER):
        d = wid * D_PER_WORKER + k
        dp, dq = d // 8, d % 8
        pltpu.sync_copy(table_hbm.at[dp, :, dq], row_v)

        pltpu.async_copy(idx_src(0), ib0, si0)
        pltpu.async_copy(idx_src(1), ib1, si1)

        def m_body(m, carry, dp=dp, dq=dq):
            for b in range(2):
                l = 2 * m + b
                ib, isem = ibufs[b], isems[b]
                ob, osem = obufs[b], osems[b]
                pltpu.make_async_copy(idx_src(l), ib, isem).wait()
                @pl.when(m > 0)
                def _():
                    pltpu.make_async_copy(
                        ob, out_dst(l - 2, dp, dq), osem
                    ).wait()

                @plsc.parallel_loop(0, 32, unroll=4)
                def _(g, ib=ib, ob=ob):
                    for j in range(8):
                        v16 = ib[g, pl.ds(j * 16, 16)]
                        h16 = lax.shift_right_logical(v16, 7)
                        l16 = lax.bitwise_and(v16, 127)
                        ob[g, pl.ds(j * 16, 16)] = plsc.load_gather(
                            row_v, [h16, l16]
                        )
                pltpu.async_copy(ob, out_dst(l, dp, dq), osem)

                @pl.when(l + 2 < MAX_LEN)
                def _():
                    pltpu.async_copy(idx_src(l + 2), ib, isem)

            return carry

        lax.fori_loop(0, MAX_LEN // 2, m_body, 0)
        pltpu.make_async_copy(ob0, out_dst(MAX_LEN - 2, dp, dq), so0).wait()
        pltpu.make_async_copy(ob1, out_dst(MAX_LEN - 1, dp, dq), so1).wait()


def kernel(indices, table):
    idx4 = indices.T.reshape(25, 8, 32, 128).transpose(0, 2, 1, 3)
    table_t = jnp.pad(table.T, ((0, 0), (0, VOCAB_PAD - VOCAB)))
    table4 = table_t.reshape(8, 8, 782, 128).transpose(0, 2, 1, 3)
    out5 = _lookup_t(idx4, table4)
    out = out5.transpose(2, 4, 0, 1, 3).reshape(BATCH, MAX_LEN, EMBED_DIM)
    return out


# speedup vs baseline: 3.4710x; 1.2992x over previous
import functools

import jax
import jax.numpy as jnp
from jax import lax
from jax.experimental import pallas as pl
from jax.experimental.pallas import tpu as pltpu
from jax.experimental.pallas import tpu_sc as plsc

VOCAB = 100000
VOCAB_PAD = 100096
EMBED_DIM = 64
BATCH = 4096
MAX_LEN = 200

NUM_WORKERS = 32
D_PER_WORKER = EMBED_DIM // NUM_WORKERS
N_PHASES = 25

_MESH = plsc.VectorSubcoreMesh(core_axis_name="c", subcore_axis_name="s")


@functools.partial(
    pl.kernel,
    mesh=_MESH,
    out_type=jax.ShapeDtypeStruct((MAX_LEN, 8, 32, 8, 128), jnp.float32),
    scratch_types=[
        pltpu.VMEM((782, 128), jnp.float32),
        pltpu.VMEM((32, 128), jnp.int32),
        pltpu.VMEM((32, 128), jnp.int32),
        pltpu.VMEM((32, 128), jnp.float32),
        pltpu.VMEM((32, 128), jnp.float32),
        pltpu.VMEM_SHARED((32, 8, 128), jnp.int32),
        pltpu.VMEM_SHARED((32, 8, 128), jnp.int32),
        pltpu.SemaphoreType.DMA,
        pltpu.SemaphoreType.DMA,
        pltpu.SemaphoreType.DMA,
        pltpu.SemaphoreType.DMA,
        pltpu.SemaphoreType.DMA,
    ],
    compiler_params=pltpu.CompilerParams(
        use_tc_tiling_on_sc=False, needs_layout_passes=False
    ),
)
def _lookup_t(
    idx_hbm, table_hbm, out_hbm,
    row_v, ib0, ib1, ob0, ob1, sha, shb,
    si0, si1, so0, so1, ssem,
):
    cid = lax.axis_index("c")
    sid = lax.axis_index("s")
    wid = sid * 2 + cid

    ibufs = (ib0, ib1)
    isems = (si0, si1)
    obufs = (ob0, ob1)
    osems = (so0, so1)
    shbufs = (sha, shb)

    def stage_start(p, buf):
        @pl.when(sid == 0)
        def _():
            pltpu.async_copy(idx_hbm.at[p], buf, ssem)

    def stage_wait(p, buf):
        @pl.when(sid == 0)
        def _():
            pltpu.make_async_copy(idx_hbm.at[p], buf, ssem).wait()

    def out_dst(l, dp, dq):
        return out_hbm.at[l, dp, :, dq]

    def sweep8(buf, p, dp, dq):
        def idx_src(ll):
            return buf.at[:, ll, :]

        pltpu.async_copy(idx_src(0), ib0, si0)
        pltpu.async_copy(idx_src(1), ib1, si1)

        def m_body(m, carry):
            for b in range(2):
                ll = 2 * m + b
                l = 8 * p + ll
                ib, isem = ibufs[b], isems[b]
                ob, osem = obufs[b], osems[b]
                pltpu.make_async_copy(idx_src(ll), ib, isem).wait()

                @pl.when(l >= 2)
                def _():
                    pltpu.make_async_copy(
                        ob, out_dst(l - 2, dp, dq), osem
                    ).wait()

                @plsc.parallel_loop(0, 32, unroll=4)
                def _(g, ib=ib, ob=ob):
                    for j in range(8):
                        v16 = ib[g, pl.ds(j * 16, 16)]
                        h16 = lax.shift_right_logical(v16, 7)
                        l16 = lax.bitwise_and(v16, 127)
                        ob[g, pl.ds(j * 16, 16)] = plsc.load_gather(
                            row_v, [h16, l16]
                        )

                pltpu.async_copy(ob, out_dst(l, dp, dq), osem)

                @pl.when(ll + 2 < 8)
                def _():
                    pltpu.async_copy(idx_src(ll + 2), ib, isem)

            return carry

        lax.fori_loop(0, 4, m_body, 0)

    def d_body(k, carry):
        d = wid * D_PER_WORKER + k
        dp, dq = d // 8, d % 8
        pltpu.sync_copy(table_hbm.at[dp, :, dq], row_v)

        stage_start(0, shbufs[0])
        stage_wait(0, shbufs[0])
        plsc.subcore_barrier()

        def p_body(p, carry2):
            for parity in range(2):
                @pl.when(p % 2 == parity)
                def _(parity=parity):
                    buf = shbufs[parity]
                    nxt = shbufs[1 - parity]

                    @pl.when(p + 1 < N_PHASES)
                    def _():
                        stage_start(p + 1, nxt)

                    sweep8(buf, p, dp, dq)

                    @pl.when(p + 1 < N_PHASES)
                    def _():
                        stage_wait(p + 1, nxt)

            plsc.subcore_barrier()
            return carry2

        lax.fori_loop(0, N_PHASES, p_body, 0)
        pltpu.make_async_copy(ob0, out_dst(MAX_LEN - 2, dp, dq), so0).wait()
        pltpu.make_async_copy(ob1, out_dst(MAX_LEN - 1, dp, dq), so1).wait()
        return carry

    lax.fori_loop(0, D_PER_WORKER, d_body, 0)


def kernel(indices, table):
    idx4 = indices.T.reshape(25, 8, 32, 128).transpose(0, 2, 1, 3)
    table_t = jnp.pad(table.T, ((0, 0), (0, VOCAB_PAD - VOCAB)))
    table4 = table_t.reshape(8, 8, 782, 128).transpose(0, 2, 1, 3)
    out5 = _lookup_t(idx4, table4)
    out = out5.transpose(2, 4, 0, 1, 3).reshape(BATCH, MAX_LEN, EMBED_DIM)
    return out


# speedup vs baseline: 3.6803x; 1.0603x over previous
import functools

import jax
import jax.numpy as jnp
from jax import lax
from jax.experimental import pallas as pl
from jax.experimental.pallas import tpu as pltpu
from jax.experimental.pallas import tpu_sc as plsc

VOCAB = 100000
VOCAB_PAD = 100096
EMBED_DIM = 64
BATCH = 4096
MAX_LEN = 200

NUM_WORKERS = 32
D_PER_WORKER = EMBED_DIM // NUM_WORKERS
N_PHASES = 25

_MESH = plsc.VectorSubcoreMesh(core_axis_name="c", subcore_axis_name="s")


@functools.partial(
    pl.kernel,
    mesh=_MESH,
    out_type=jax.ShapeDtypeStruct((MAX_LEN, 8, 32, 8, 128), jnp.float32),
    scratch_types=[
        pltpu.VMEM((782, 128), jnp.float32),
        pltpu.VMEM((2, 32, 128), jnp.int32),
        pltpu.VMEM((2, 32, 128), jnp.int32),
        pltpu.VMEM((32, 128), jnp.float32),
        pltpu.VMEM((32, 128), jnp.float32),
        pltpu.VMEM_SHARED((8, 32, 128), jnp.int32),
        pltpu.VMEM_SHARED((8, 32, 128), jnp.int32),
        pltpu.SemaphoreType.DMA,
        pltpu.SemaphoreType.DMA,
        pltpu.SemaphoreType.DMA,
        pltpu.SemaphoreType.DMA,
        pltpu.SemaphoreType.DMA,
    ],
    compiler_params=pltpu.CompilerParams(
        use_tc_tiling_on_sc=False, needs_layout_passes=False
    ),
)
def _lookup_t(
    idx_hbm, table_hbm, out_hbm,
    row_v, ib0, ib1, ob0, ob1, sha, shb,
    si0, si1, so0, so1, ssem,
):
    cid = lax.axis_index("c")
    sid = lax.axis_index("s")
    wid = sid * 2 + cid

    ibufs = (ib0, ib1)
    isems = (si0, si1)
    obufs = (ob0, ob1)
    osems = (so0, so1)
    shbufs = (sha, shb)

    def stage_start(p, buf):
        @pl.when(sid == 0)
        def _():
            pltpu.async_copy(idx_hbm.at[p], buf, ssem)

    def stage_wait(p, buf):
        @pl.when(sid == 0)
        def _():
            pltpu.make_async_copy(idx_hbm.at[p], buf, ssem).wait()

    def out_dst(l, dp, dq):
        return out_hbm.at[l, dp, :, dq]

    def sweep8(buf, p, dp, dq):
        def idx_src(j):
            return buf.at[pl.ds(2 * j, 2)]

        pltpu.async_copy(idx_src(0), ib0, si0)
        pltpu.async_copy(idx_src(1), ib1, si1)

        def m_body(j, carry):
            jb = j % 2
            for jbs in range(2):
                @pl.when(jb == jbs)
                def _(jbs=jbs):
                    ib, isem = ibufs[jbs], isems[jbs]
                    pltpu.make_async_copy(idx_src(j), ib, isem).wait()
                    for i in range(2):
                        ll = 2 * j + i
                        l = 8 * p + ll
                        ob, osem = obufs[i], osems[i]

                        @pl.when(l >= 2)
                        def _():
                            pltpu.make_async_copy(
                                ob, out_dst(l - 2, dp, dq), osem
                            ).wait()

                        @plsc.parallel_loop(0, 32, unroll=4)
                        def _(g, ib=ib, ob=ob, i=i):
                            for u in range(8):
                                v16 = ib[i, g, pl.ds(u * 16, 16)]
                                h16 = lax.shift_right_logical(v16, 7)
                                l16 = lax.bitwise_and(v16, 127)
                                ob[g, pl.ds(u * 16, 16)] = plsc.load_gather(
                                    row_v, [h16, l16]
                                )

                        pltpu.async_copy(ob, out_dst(l, dp, dq), osem)

                    @pl.when(j + 2 < 4)
                    def _():
                        pltpu.async_copy(idx_src(j + 2), ib, isem)

            return carry

        lax.fori_loop(0, 4, m_body, 0)

    def d_body(k, carry):
        d = wid * D_PER_WORKER + k
        dp, dq = d // 8, d % 8
        pltpu.sync_copy(table_hbm.at[dp, :, dq], row_v)

        stage_start(0, shbufs[0])
        stage_wait(0, shbufs[0])
        plsc.subcore_barrier()

        def p_body(p, carry2):
            for parity in range(2):
                @pl.when(p % 2 == parity)
                def _(parity=parity):
                    buf = shbufs[parity]
                    nxt = shbufs[1 - parity]

                    @pl.when(p + 1 < N_PHASES)
                    def _():
                        stage_start(p + 1, nxt)

                    sweep8(buf, p, dp, dq)

                    @pl.when(p + 1 < N_PHASES)
                    def _():
                        stage_wait(p + 1, nxt)

            plsc.subcore_barrier()
            return carry2

        lax.fori_loop(0, N_PHASES, p_body, 0)
        pltpu.make_async_copy(ob0, out_dst(MAX_LEN - 2, dp, dq), so0).wait()
        pltpu.make_async_copy(ob1, out_dst(MAX_LEN - 1, dp, dq), so1).wait()
        return carry

    lax.fori_loop(0, D_PER_WORKER, d_body, 0)


def kernel(indices, table):
    idx4 = indices.T.reshape(25, 8, 32, 128)
    table_t = jnp.pad(table.T, ((0, 0), (0, VOCAB_PAD - VOCAB)))
    table4 = table_t.reshape(8, 8, 782, 128).transpose(0, 2, 1, 3)
    out5 = _lookup_t(idx4, table4)
    out = out5.transpose(2, 4, 0, 1, 3).reshape(BATCH, MAX_LEN, EMBED_DIM)
    return out


# speedup vs baseline: 3.8866x; 1.0561x over previous
import functools

import jax
import jax.numpy as jnp
from jax import lax
from jax.experimental import pallas as pl
from jax.experimental.pallas import tpu as pltpu
from jax.experimental.pallas import tpu_sc as plsc

VOCAB = 100000
VOCAB_PAD = 100096
EMBED_DIM = 64
BATCH = 4096
MAX_LEN = 200

NUM_WORKERS = 32
D_PER_WORKER = EMBED_DIM // NUM_WORKERS
PHASE_L = 12
N_FULL_PHASES = 16

_MESH = plsc.VectorSubcoreMesh(core_axis_name="c", subcore_axis_name="s")


@functools.partial(
    pl.kernel,
    mesh=_MESH,
    out_type=jax.ShapeDtypeStruct((MAX_LEN, 8, 32, 8, 128), jnp.float32),
    scratch_types=[
        pltpu.VMEM((782, 128), jnp.float32),
        pltpu.VMEM((2, 32, 128), jnp.int32),
        pltpu.VMEM((2, 32, 128), jnp.int32),
        pltpu.VMEM((32, 128), jnp.float32),
        pltpu.VMEM((32, 128), jnp.float32),
        pltpu.VMEM_SHARED((12, 32, 128), jnp.int32),
        pltpu.VMEM_SHARED((12, 32, 128), jnp.int32),
        pltpu.SemaphoreType.DMA,
        pltpu.SemaphoreType.DMA,
        pltpu.SemaphoreType.DMA,
        pltpu.SemaphoreType.DMA,
        pltpu.SemaphoreType.DMA,
    ],
    compiler_params=pltpu.CompilerParams(
        use_tc_tiling_on_sc=False, needs_layout_passes=False
    ),
)
def _lookup_t(
    idx_hbm, table_hbm, out_hbm,
    row_v, ib0, ib1, ob0, ob1, sha, shb,
    si0, si1, so0, so1, ssem,
):
    cid = lax.axis_index("c")
    sid = lax.axis_index("s")
    wid = sid * 2 + cid

    ibufs = (ib0, ib1)
    isems = (si0, si1)
    obufs = (ob0, ob1)
    osems = (so0, so1)
    shbufs = (sha, shb)

    def stage2_start(p, buf):
        @pl.when(sid == 0)
        def _():
            pltpu.async_copy(idx_hbm.at[pl.ds(PHASE_L * p, PHASE_L)], buf, ssem)

    def stage2_wait(p, buf):
        @pl.when(sid == 0)
        def _():
            pltpu.make_async_copy(
                idx_hbm.at[pl.ds(PHASE_L * p, PHASE_L)], buf, ssem
            ).wait()

    def stage1_start(buf):
        @pl.when(sid == 0)
        def _():
            pltpu.async_copy(
                idx_hbm.at[pl.ds(192, 8)], buf.at[pl.ds(0, 8)], ssem
            )

    def stage1_wait(buf):
        @pl.when(sid == 0)
        def _():
            pltpu.make_async_copy(
                idx_hbm.at[pl.ds(192, 8)], buf.at[pl.ds(0, 8)], ssem
            ).wait()

    def out_dst(l, dp, dq):
        return out_hbm.at[l, dp, :, dq]

    def sweep(buf, n_pairs, l0, dp, dq):
        def idx_src(j):
            return buf.at[pl.ds(2 * j, 2)]

        pltpu.async_copy(idx_src(0), ib0, si0)
        pltpu.async_copy(idx_src(1), ib1, si1)

        def m_body(j, carry):
            jb = j % 2
            for jbs in range(2):
                @pl.when(jb == jbs)
                def _(jbs=jbs):
                    ib, isem = ibufs[jbs], isems[jbs]
                    pltpu.make_async_copy(idx_src(j), ib, isem).wait()
                    for i in range(2):
                        ll = 2 * j + i
                        l = l0 + ll
                        ob, osem = obufs[i], osems[i]

                        @pl.when(l >= 2)
                        def _():
                            pltpu.make_async_copy(
                                ob, out_dst(l - 2, dp, dq), osem
                            ).wait()

                        @plsc.parallel_loop(0, 32, unroll=4)
                        def _(g, ib=ib, ob=ob, i=i):
                            for u in range(8):
                                v16 = ib[i, g, pl.ds(u * 16, 16)]
                                h16 = lax.shift_right_logical(v16, 7)
                                l16 = lax.bitwise_and(v16, 127)
                                ob[g, pl.ds(u * 16, 16)] = plsc.load_gather(
                                    row_v, [h16, l16]
                                )

                        pltpu.async_copy(ob, out_dst(l, dp, dq), osem)

                    @pl.when(j + 2 < n_pairs)
                    def _():
                        pltpu.async_copy(idx_src(j + 2), ib, isem)

            return carry

        lax.fori_loop(0, n_pairs, m_body, 0)

    def d_body(k, carry):
        d = wid * D_PER_WORKER + k
        dp, dq = d // 8, d % 8
        pltpu.sync_copy(table_hbm.at[dp, :, dq], row_v)

        stage2_start(0, shbufs[0])
        stage2_wait(0, shbufs[0])
        plsc.subcore_barrier()

        def p_body(p, carry2):
            for parity in range(2):
                @pl.when(p % 2 == parity)
                def _(parity=parity):
                    buf = shbufs[parity]
                    nxt = shbufs[1 - parity]

                    @pl.when(p + 1 < N_FULL_PHASES)
                    def _():
                        stage2_start(p + 1, nxt)

                    @pl.when(p + 1 == N_FULL_PHASES)
                    def _():
                        stage1_start(nxt)

                    sweep(buf, PHASE_L // 2, PHASE_L * p, dp, dq)

                    @pl.when(p + 1 < N_FULL_PHASES)
                    def _():
                        stage2_wait(p + 1, nxt)

                    @pl.when(p + 1 == N_FULL_PHASES)
                    def _():
                        stage1_wait(nxt)

            plsc.subcore_barrier()
            return carry2

        lax.fori_loop(0, N_FULL_PHASES, p_body, 0)
        sweep(shbufs[0], 4, 192, dp, dq)
        plsc.subcore_barrier()
        pltpu.make_async_copy(ob0, out_dst(MAX_LEN - 2, dp, dq), so0).wait()
        pltpu.make_async_copy(ob1, out_dst(MAX_LEN - 1, dp, dq), so1).wait()
        return carry

    lax.fori_loop(0, D_PER_WORKER, d_body, 0)


def kernel(indices, table):
    idx4 = indices.T.reshape(200, 32, 128)
    table_t = jnp.pad(table.T, ((0, 0), (0, VOCAB_PAD - VOCAB)))
    table4 = table_t.reshape(8, 8, 782, 128).transpose(0, 2, 1, 3)
    out5 = _lookup_t(idx4, table4)
    out = out5.transpose(2, 4, 0, 1, 3).reshape(BATCH, MAX_LEN, EMBED_DIM)
    return out
